# Initial kernel scaffold; baseline (speedup 1.0000x reference)
import jax
import jax.numpy as jnp
from jax.experimental import pallas as pl


def kernel(hidden_states, gate_w, w1, w2):
    raise NotImplementedError("write your pallas kernel here")


# speedup vs baseline: 1.3301x; 1.3301x over previous
import functools

import jax
import jax.numpy as jnp
from jax.experimental import pallas as pl
from jax.experimental.pallas import tpu as pltpu

E = 8
TOPK = 2
D = 1024
F = 2048
T = 2048

FC = 512
J = F // FC


def _router_body(x_ref, gw_ref, wd_ref):
    x = x_ref[...]
    logits = jax.lax.dot_general(
        x, gw_ref[...], (((1,), (1,)), ((), ())),
        preferred_element_type=jnp.float32)
    m = jnp.max(logits, axis=-1, keepdims=True)
    ex = jnp.exp(logits - m)
    probs = ex / jnp.sum(ex, axis=-1, keepdims=True)

    iota = jax.lax.broadcasted_iota(jnp.int32, (T, E), 1)
    p0 = jnp.max(probs, axis=-1, keepdims=True)
    i0 = jnp.min(jnp.where(probs == p0, iota, E), axis=-1, keepdims=True)
    masked = jnp.where(iota == i0, -1.0, probs)
    p1 = jnp.max(masked, axis=-1, keepdims=True)
    i1 = jnp.min(jnp.where(masked == p1, iota, E), axis=-1, keepdims=True)
    s = p0 + p1
    w0 = p0 / s
    w1 = p1 / s
    wd = jnp.where(iota == i0, w0, 0.0) + jnp.where(iota == i1, w1, 0.0)
    wd_ref[...] = wd


def _expert_body(x_ref, wd_ref, w1g_ref, w1u_ref, w2_ref, out_ref):
    e = pl.program_id(0)
    j = pl.program_id(1)
    x = x_ref[...]
    g = jax.lax.dot_general(
        x, w1g_ref[0], (((1,), (1,)), ((), ())),
        preferred_element_type=jnp.float32)
    u = jax.lax.dot_general(
        x, w1u_ref[0], (((1,), (1,)), ((), ())),
        preferred_element_type=jnp.float32)
    act = g * jax.nn.sigmoid(g) * u
    y = jax.lax.dot_general(
        act, w2_ref[0], (((1,), (1,)), ((), ())),
        preferred_element_type=jnp.float32)
    onehot = (jax.lax.broadcasted_iota(jnp.int32, (E, 1), 0) == e
              ).astype(jnp.float32)
    wcol = jnp.dot(wd_ref[...], onehot,
                   preferred_element_type=jnp.float32)
    contrib = y * wcol

    @pl.when(jnp.logical_and(e == 0, j == 0))
    def _init():
        out_ref[...] = contrib

    @pl.when(jnp.logical_not(jnp.logical_and(e == 0, j == 0)))
    def _acc():
        out_ref[...] += contrib


@jax.jit
def kernel(hidden_states, gate_w, w1, w2):
    wd = pl.pallas_call(
        _router_body,
        out_shape=jax.ShapeDtypeStruct((T, E), jnp.float32),
    )(hidden_states, gate_w)

    out = pl.pallas_call(
        _expert_body,
        grid=(E, J),
        in_specs=[
            pl.BlockSpec((T, D), lambda e, j: (0, 0)),
            pl.BlockSpec((T, E), lambda e, j: (0, 0)),
            pl.BlockSpec((1, FC, D), lambda e, j: (e, j, 0)),
            pl.BlockSpec((1, FC, D), lambda e, j: (e, J + j, 0)),
            pl.BlockSpec((1, D, FC), lambda e, j: (e, 0, j)),
        ],
        out_specs=pl.BlockSpec((T, D), lambda e, j: (0, 0)),
        out_shape=jax.ShapeDtypeStruct((T, D), jnp.float32),
        compiler_params=pltpu.CompilerParams(
            dimension_semantics=("arbitrary", "arbitrary")),
    )(hidden_states, wd, w1, w1, w2)
    return out
